# Initial kernel scaffold; baseline (speedup 1.0000x reference)
import jax
import jax.numpy as jnp
from jax.experimental import pallas as pl


def kernel(x, edge_index, W1, b1, W2, b2):
    raise NotImplementedError("write your pallas kernel here")


# speedup vs baseline: 43.5890x; 43.5890x over previous
import functools

import jax
import jax.numpy as jnp
from jax import lax
from jax.experimental import pallas as pl
from jax.experimental.pallas import tpu as pltpu


def _pad_to(v, m):
    return (v + m - 1) // m * m


def _t1_body(x_ref, w1_ref, b1_ref, out_ref):
    h = lax.dot_general(x_ref[:], w1_ref[:], (((1,), (1,)), ((), ())),
                        preferred_element_type=jnp.float32)
    out_ref[:] = jnp.maximum(h + b1_ref[:], 0.0)


def _t1(x_pad, W1, b1, bm):
    np_, d_feat = x_pad.shape
    hid = W1.shape[0]
    return pl.pallas_call(
        _t1_body,
        grid=(np_ // bm,),
        in_specs=[
            pl.BlockSpec((bm, d_feat), lambda i: (i, 0)),
            pl.BlockSpec((hid, d_feat), lambda i: (0, 0)),
            pl.BlockSpec((1, hid), lambda i: (0, 0)),
        ],
        out_specs=pl.BlockSpec((bm, hid), lambda i: (i, 0)),
        out_shape=jax.ShapeDtypeStruct((np_, hid), jnp.float32),
    )(x_pad, W1, b1.reshape(1, hid))


def _t2_body(aik_ref, akj_ref, aij_ref, m2_ref, m1_ref, cs1_ref, cs2_ref,
             acc_ref, *, nk, bm, bn):
    j = pl.program_id(0)
    i = pl.program_id(1)
    k = pl.program_id(2)

    @pl.when(k == 0)
    def _():
        acc_ref[:] = jnp.zeros_like(acc_ref)

    acc_ref[:] += lax.dot_general(
        aik_ref[:], akj_ref[:], (((1,), (0,)), ((), ())),
        preferred_element_type=jnp.float32)

    @pl.when(k == nk - 1)
    def _():
        adj = aij_ref[:] != 0
        rows = i * bm + lax.broadcasted_iota(jnp.int32, (bm, bn), 0)
        cols = j * bn + lax.broadcasted_iota(jnp.int32, (bm, bn), 1)
        notdiag = rows != cols
        m1 = adj & notdiag
        m2 = (acc_ref[:] > 0) & (~adj) & notdiag
        m1_ref[:] = m1.astype(jnp.int8)
        m2_ref[:] = m2.astype(jnp.int8)
        c1 = jnp.sum(m1.astype(jnp.float32), axis=0, keepdims=True)
        c2 = jnp.sum(m2.astype(jnp.float32), axis=0, keepdims=True)

        @pl.when(i == 0)
        def _():
            cs1_ref[:] = c1
            cs2_ref[:] = c2

        @pl.when(i > 0)
        def _():
            cs1_ref[:] += c1
            cs2_ref[:] += c2


def _t2(A, bm, bn, bk):
    np_ = A.shape[0]
    nj, ni, nk = np_ // bn, np_ // bm, np_ // bk
    body = functools.partial(_t2_body, nk=nk, bm=bm, bn=bn)
    return pl.pallas_call(
        body,
        grid=(nj, ni, nk),
        in_specs=[
            pl.BlockSpec((bm, bk), lambda j, i, k: (i, k)),
            pl.BlockSpec((bk, bn), lambda j, i, k: (k, j)),
            pl.BlockSpec((bm, bn), lambda j, i, k: (i, j)),
        ],
        out_specs=[
            pl.BlockSpec((bm, bn), lambda j, i, k: (i, j)),
            pl.BlockSpec((bm, bn), lambda j, i, k: (i, j)),
            pl.BlockSpec((1, bn), lambda j, i, k: (0, j)),
            pl.BlockSpec((1, bn), lambda j, i, k: (0, j)),
        ],
        out_shape=[
            jax.ShapeDtypeStruct((np_, np_), jnp.int8),
            jax.ShapeDtypeStruct((np_, np_), jnp.int8),
            jax.ShapeDtypeStruct((1, np_), jnp.float32),
            jax.ShapeDtypeStruct((1, np_), jnp.float32),
        ],
        scratch_shapes=[pltpu.VMEM((bm, bn), jnp.float32)],
        compiler_params=pltpu.CompilerParams(
            dimension_semantics=("arbitrary", "arbitrary", "arbitrary")),
    )(A, A, A)


def _t3_body(m1_ref, m2_ref, r_ref, d1k_ref, d2k_ref, d1i_ref, d2i_ref,
             out_ref, acc1_ref, acc2_ref, *, nk):
    k = pl.program_id(1)

    @pl.when(k == 0)
    def _():
        acc1_ref[:] = jnp.zeros_like(acc1_ref)
        acc2_ref[:] = jnp.zeros_like(acc2_ref)

    rd1 = r_ref[:] * d1k_ref[:]
    rd2 = r_ref[:] * d2k_ref[:]
    acc1_ref[:] += lax.dot_general(
        m1_ref[:].astype(jnp.float32), rd1, (((1,), (0,)), ((), ())),
        preferred_element_type=jnp.float32)
    acc2_ref[:] += lax.dot_general(
        m2_ref[:].astype(jnp.float32), rd2, (((1,), (0,)), ((), ())),
        preferred_element_type=jnp.float32)

    @pl.when(k == nk - 1)
    def _():
        out_ref[:] = jnp.maximum(d1i_ref[:] * acc1_ref[:],
                                 d2i_ref[:] * acc2_ref[:])


def _t3(m1, m2, R, d1c, d2c, bm, bk):
    np_, hid = R.shape
    ni, nk = np_ // bm, np_ // bk
    body = functools.partial(_t3_body, nk=nk)
    return pl.pallas_call(
        body,
        grid=(ni, nk),
        in_specs=[
            pl.BlockSpec((bm, bk), lambda i, k: (i, k)),
            pl.BlockSpec((bm, bk), lambda i, k: (i, k)),
            pl.BlockSpec((bk, hid), lambda i, k: (k, 0)),
            pl.BlockSpec((bk, 1), lambda i, k: (k, 0)),
            pl.BlockSpec((bk, 1), lambda i, k: (k, 0)),
            pl.BlockSpec((bm, 1), lambda i, k: (i, 0)),
            pl.BlockSpec((bm, 1), lambda i, k: (i, 0)),
        ],
        out_specs=pl.BlockSpec((bm, hid), lambda i, k: (i, 0)),
        out_shape=jax.ShapeDtypeStruct((np_, hid), jnp.float32),
        scratch_shapes=[pltpu.VMEM((bm, hid), jnp.float32),
                        pltpu.VMEM((bm, hid), jnp.float32)],
        compiler_params=pltpu.CompilerParams(
            dimension_semantics=("arbitrary", "arbitrary")),
    )(m1, m2, R, d1c, d2c, d1c, d2c)


def _t4_body(r0_ref, r1_ref, r2_ref, w0_ref, w1_ref, w2_ref, b_ref, out_ref):
    logits = (
        lax.dot_general(r0_ref[:], w0_ref[:], (((1,), (0,)), ((), ())),
                        preferred_element_type=jnp.float32)
        + lax.dot_general(r1_ref[:], w1_ref[:], (((1,), (0,)), ((), ())),
                          preferred_element_type=jnp.float32)
        + lax.dot_general(r2_ref[:], w2_ref[:], (((1,), (0,)), ((), ())),
                          preferred_element_type=jnp.float32)
        + b_ref[:])
    m = jnp.max(logits, axis=1, keepdims=True)
    s = logits - m
    lse = jnp.log(jnp.sum(jnp.exp(s), axis=1, keepdims=True))
    out_ref[:] = s - lse


def _t4(R0, R1, R2, W2, b2, bm):
    np_, hid = R0.shape
    ncls = W2.shape[0]
    w0 = W2[:, 0 * hid:1 * hid].T
    w1 = W2[:, 1 * hid:2 * hid].T
    w2 = W2[:, 2 * hid:3 * hid].T
    return pl.pallas_call(
        _t4_body,
        grid=(np_ // bm,),
        in_specs=[
            pl.BlockSpec((bm, hid), lambda i: (i, 0)),
            pl.BlockSpec((bm, hid), lambda i: (i, 0)),
            pl.BlockSpec((bm, hid), lambda i: (i, 0)),
            pl.BlockSpec((hid, ncls), lambda i: (0, 0)),
            pl.BlockSpec((hid, ncls), lambda i: (0, 0)),
            pl.BlockSpec((hid, ncls), lambda i: (0, 0)),
            pl.BlockSpec((1, ncls), lambda i: (0, 0)),
        ],
        out_specs=pl.BlockSpec((bm, ncls), lambda i: (i, 0)),
        out_shape=jax.ShapeDtypeStruct((np_, ncls), jnp.float32),
    )(R0, R1, R2, w0, w1, w2, b2.reshape(1, ncls))


def kernel(x, edge_index, W1, b1, W2, b2):
    n, d_feat = x.shape
    hid = W1.shape[0]
    np_ = _pad_to(n, 1024)
    bm2 = bn2 = bk2 = min(1024, np_)
    bm3 = min(512, np_)
    bk3 = min(2048, np_)
    bm14 = min(1024, np_)

    src = edge_index[0].astype(jnp.int32)
    dst = edge_index[1].astype(jnp.int32)

    A = jnp.zeros((np_, np_), jnp.bfloat16).at[src, dst].set(jnp.bfloat16(1))

    m2, m1, cs1, cs2 = _t2(A, bm2, bn2, bk2)
    d1 = jnp.where(cs1 > 0, lax.rsqrt(jnp.maximum(cs1, 1e-30)), 0.0)
    d2 = jnp.where(cs2 > 0, lax.rsqrt(jnp.maximum(cs2, 1e-30)), 0.0)
    d1c = d1.reshape(np_, 1)
    d2c = d2.reshape(np_, 1)

    x_pad = jnp.pad(x, ((0, np_ - n), (0, 0)))
    R0 = _t1(x_pad, W1, b1, bm14)
    R1 = _t3(m1, m2, R0, d1c, d2c, bm3, bk3)
    R2 = _t3(m1, m2, R1, d1c, d2c, bm3, bk3)

    out = _t4(R0, R1, R2, W2, b2, bm14)
    return out[:n]


# speedup vs baseline: 47.5362x; 1.0906x over previous
import functools

import jax
import jax.numpy as jnp
from jax import lax
from jax.experimental import pallas as pl
from jax.experimental.pallas import tpu as pltpu


def _pad_to(v, m):
    return (v + m - 1) // m * m


def _t1_body(x_ref, w1_ref, b1_ref, out_ref):
    h = lax.dot_general(x_ref[:], w1_ref[:], (((1,), (1,)), ((), ())),
                        preferred_element_type=jnp.float32)
    out_ref[:] = jnp.maximum(h + b1_ref[:], 0.0)


def _t1(x_pad, W1, b1, bm):
    np_, d_feat = x_pad.shape
    hid = W1.shape[0]
    return pl.pallas_call(
        _t1_body,
        grid=(np_ // bm,),
        in_specs=[
            pl.BlockSpec((bm, d_feat), lambda i: (i, 0)),
            pl.BlockSpec((hid, d_feat), lambda i: (0, 0)),
            pl.BlockSpec((1, hid), lambda i: (0, 0)),
        ],
        out_specs=pl.BlockSpec((bm, hid), lambda i: (i, 0)),
        out_shape=jax.ShapeDtypeStruct((np_, hid), jnp.float32),
    )(x_pad, W1, b1.reshape(1, hid))


def _t2_body(aik_ref, akj_ref, aij_ref, m2_ref, m1_ref, cs1_ref, cs2_ref,
             acc_ref, *, nk, bm, bn):
    j = pl.program_id(0)
    i = pl.program_id(1)
    k = pl.program_id(2)

    @pl.when(k == 0)
    def _():
        acc_ref[:] = jnp.zeros_like(acc_ref)

    acc_ref[:] += lax.dot_general(
        aik_ref[:], akj_ref[:], (((1,), (0,)), ((), ())),
        preferred_element_type=jnp.int32)

    @pl.when(k == nk - 1)
    def _():
        adj = aij_ref[:] != 0
        rows = i * bm + lax.broadcasted_iota(jnp.int32, (bm, bn), 0)
        cols = j * bn + lax.broadcasted_iota(jnp.int32, (bm, bn), 1)
        notdiag = rows != cols
        m1 = adj & notdiag
        m2 = (acc_ref[:] > 0) & (~adj) & notdiag
        m1_ref[:] = m1.astype(jnp.int8)
        m2_ref[:] = m2.astype(jnp.int8)
        c1 = jnp.sum(m1.astype(jnp.float32), axis=0, keepdims=True)
        c2 = jnp.sum(m2.astype(jnp.float32), axis=0, keepdims=True)

        @pl.when(i == 0)
        def _():
            cs1_ref[:] = c1
            cs2_ref[:] = c2

        @pl.when(i > 0)
        def _():
            cs1_ref[:] += c1
            cs2_ref[:] += c2


def _t2(A, bm, bn, bk):
    np_ = A.shape[0]
    nj, ni, nk = np_ // bn, np_ // bm, np_ // bk
    body = functools.partial(_t2_body, nk=nk, bm=bm, bn=bn)
    return pl.pallas_call(
        body,
        grid=(nj, ni, nk),
        in_specs=[
            pl.BlockSpec((bm, bk), lambda j, i, k: (i, k)),
            pl.BlockSpec((bk, bn), lambda j, i, k: (k, j)),
            pl.BlockSpec((bm, bn), lambda j, i, k: (i, j)),
        ],
        out_specs=[
            pl.BlockSpec((bm, bn), lambda j, i, k: (i, j)),
            pl.BlockSpec((bm, bn), lambda j, i, k: (i, j)),
            pl.BlockSpec((1, bn), lambda j, i, k: (0, j)),
            pl.BlockSpec((1, bn), lambda j, i, k: (0, j)),
        ],
        out_shape=[
            jax.ShapeDtypeStruct((np_, np_), jnp.int8),
            jax.ShapeDtypeStruct((np_, np_), jnp.int8),
            jax.ShapeDtypeStruct((1, np_), jnp.float32),
            jax.ShapeDtypeStruct((1, np_), jnp.float32),
        ],
        scratch_shapes=[pltpu.VMEM((bm, bn), jnp.int32)],
        compiler_params=pltpu.CompilerParams(
            dimension_semantics=("arbitrary", "arbitrary", "arbitrary")),
    )(A, A, A)


def _t3_body(m1_ref, m2_ref, r_ref, d1k_ref, d2k_ref, d1i_ref, d2i_ref,
             out_ref, acc1_ref, acc2_ref, *, nk):
    k = pl.program_id(1)

    @pl.when(k == 0)
    def _():
        acc1_ref[:] = jnp.zeros_like(acc1_ref)
        acc2_ref[:] = jnp.zeros_like(acc2_ref)

    rd1 = r_ref[:] * d1k_ref[:]
    rd2 = r_ref[:] * d2k_ref[:]
    acc1_ref[:] += lax.dot_general(
        m1_ref[:].astype(jnp.float32), rd1, (((1,), (0,)), ((), ())),
        preferred_element_type=jnp.float32)
    acc2_ref[:] += lax.dot_general(
        m2_ref[:].astype(jnp.float32), rd2, (((1,), (0,)), ((), ())),
        preferred_element_type=jnp.float32)

    @pl.when(k == nk - 1)
    def _():
        out_ref[:] = jnp.maximum(d1i_ref[:] * acc1_ref[:],
                                 d2i_ref[:] * acc2_ref[:])


def _t3(m1, m2, R, d1c, d2c, bm, bk):
    np_, hid = R.shape
    ni, nk = np_ // bm, np_ // bk
    body = functools.partial(_t3_body, nk=nk)
    return pl.pallas_call(
        body,
        grid=(ni, nk),
        in_specs=[
            pl.BlockSpec((bm, bk), lambda i, k: (i, k)),
            pl.BlockSpec((bm, bk), lambda i, k: (i, k)),
            pl.BlockSpec((bk, hid), lambda i, k: (k, 0)),
            pl.BlockSpec((bk, 1), lambda i, k: (k, 0)),
            pl.BlockSpec((bk, 1), lambda i, k: (k, 0)),
            pl.BlockSpec((bm, 1), lambda i, k: (i, 0)),
            pl.BlockSpec((bm, 1), lambda i, k: (i, 0)),
        ],
        out_specs=pl.BlockSpec((bm, hid), lambda i, k: (i, 0)),
        out_shape=jax.ShapeDtypeStruct((np_, hid), jnp.float32),
        scratch_shapes=[pltpu.VMEM((bm, hid), jnp.float32),
                        pltpu.VMEM((bm, hid), jnp.float32)],
        compiler_params=pltpu.CompilerParams(
            dimension_semantics=("arbitrary", "arbitrary")),
    )(m1, m2, R, d1c, d2c, d1c, d2c)


def _t4_body(r0_ref, r1_ref, r2_ref, w0_ref, w1_ref, w2_ref, b_ref, out_ref):
    logits = (
        lax.dot_general(r0_ref[:], w0_ref[:], (((1,), (0,)), ((), ())),
                        preferred_element_type=jnp.float32)
        + lax.dot_general(r1_ref[:], w1_ref[:], (((1,), (0,)), ((), ())),
                          preferred_element_type=jnp.float32)
        + lax.dot_general(r2_ref[:], w2_ref[:], (((1,), (0,)), ((), ())),
                          preferred_element_type=jnp.float32)
        + b_ref[:])
    m = jnp.max(logits, axis=1, keepdims=True)
    s = logits - m
    lse = jnp.log(jnp.sum(jnp.exp(s), axis=1, keepdims=True))
    out_ref[:] = s - lse


def _t4(R0, R1, R2, W2, b2, bm):
    np_, hid = R0.shape
    ncls = W2.shape[0]
    w0 = W2[:, 0 * hid:1 * hid].T
    w1 = W2[:, 1 * hid:2 * hid].T
    w2 = W2[:, 2 * hid:3 * hid].T
    return pl.pallas_call(
        _t4_body,
        grid=(np_ // bm,),
        in_specs=[
            pl.BlockSpec((bm, hid), lambda i: (i, 0)),
            pl.BlockSpec((bm, hid), lambda i: (i, 0)),
            pl.BlockSpec((bm, hid), lambda i: (i, 0)),
            pl.BlockSpec((hid, ncls), lambda i: (0, 0)),
            pl.BlockSpec((hid, ncls), lambda i: (0, 0)),
            pl.BlockSpec((hid, ncls), lambda i: (0, 0)),
            pl.BlockSpec((1, ncls), lambda i: (0, 0)),
        ],
        out_specs=pl.BlockSpec((bm, ncls), lambda i: (i, 0)),
        out_shape=jax.ShapeDtypeStruct((np_, ncls), jnp.float32),
    )(R0, R1, R2, w0, w1, w2, b2.reshape(1, ncls))


def kernel(x, edge_index, W1, b1, W2, b2):
    n, d_feat = x.shape
    hid = W1.shape[0]
    np_ = _pad_to(n, 2048)
    bm2 = bn2 = min(2048, np_)
    bk2 = min(512, np_)
    bm3 = min(512, np_)
    bk3 = min(2048, np_)
    bm14 = min(1024, np_)

    src = edge_index[0].astype(jnp.int32)
    dst = edge_index[1].astype(jnp.int32)

    A = jnp.zeros((np_, np_), jnp.int8).at[src, dst].set(jnp.int8(1))

    m2, m1, cs1, cs2 = _t2(A, bm2, bn2, bk2)
    d1 = jnp.where(cs1 > 0, lax.rsqrt(jnp.maximum(cs1, 1e-30)), 0.0)
    d2 = jnp.where(cs2 > 0, lax.rsqrt(jnp.maximum(cs2, 1e-30)), 0.0)
    d1c = d1.reshape(np_, 1)
    d2c = d2.reshape(np_, 1)

    x_pad = jnp.pad(x, ((0, np_ - n), (0, 0)))
    R0 = _t1(x_pad, W1, b1, bm14)
    R1 = _t3(m1, m2, R0, d1c, d2c, bm3, bk3)
    R2 = _t3(m1, m2, R1, d1c, d2c, bm3, bk3)

    out = _t4(R0, R1, R2, W2, b2, bm14)
    return out[:n]
